# Initial kernel scaffold; baseline (speedup 1.0000x reference)
import jax
import jax.numpy as jnp
from jax.experimental import pallas as pl


def kernel(inputs, deep_table, wide_table, W1, b1, W2, b2, W3, b3, W4, b4, lr_W, lr_b):
    raise NotImplementedError("write your pallas kernel here")


# speedup vs baseline: 17.4082x; 17.4082x over previous
import functools

import jax
import jax.numpy as jnp
import numpy as np
from jax import lax
from jax.experimental import pallas as pl
from jax.experimental.pallas import tpu as pltpu
from jax.experimental.pallas import tpu_sc as plsc

_B = 16384
_F = 26
_D = 16
_NW = 32
_TOT = _B * _F
_SUB = 128
_NROWS = _TOT // _SUB
_ROWS_PER_W = _NROWS // _NW
_K = 13
_HALF = 52


def _sc_gather_body(table_hbm, idx_hbm, out_hbm, idx_v, rows_v, sem):
    c = lax.axis_index("c")
    s = lax.axis_index("s")
    wid = s * 2 + c
    row0 = wid * _ROWS_PER_W
    for half in range(_ROWS_PER_W // _HALF):
        base = row0 + half * _HALF
        pltpu.sync_copy(idx_hbm.at[pl.ds(base * _SUB, _HALF * _SUB)], idx_v)
        for g in range(_HALF // _K):
            copies = []
            for j in range(_K):
                r = g * _K + j
                copies.append(
                    pltpu.async_copy(
                        table_hbm.at[idx_v.at[pl.ds(r * _SUB, _SUB)]],
                        rows_v.at[r], sem)
                )
            for cp in copies:
                cp.wait()
        pltpu.sync_copy(rows_v, out_hbm.at[pl.ds(base, _HALF)])


@functools.partial(
    pl.kernel,
    mesh=plsc.VectorSubcoreMesh(core_axis_name="c", subcore_axis_name="s"),
    compiler_params=pltpu.CompilerParams(use_tc_tiling_on_sc=False),
    out_type=jax.ShapeDtypeStruct((_NROWS, _SUB, _D), jnp.float32),
    scratch_types=[
        pltpu.VMEM((_HALF * _SUB,), jnp.int32),
        pltpu.VMEM((_HALF, _SUB, _D), jnp.float32),
        pltpu.SemaphoreType.DMA,
    ],
)
def _sc_gather(table_hbm, idx_hbm, out_hbm, idx_v, rows_v, sem):
    _sc_gather_body(table_hbm, idx_hbm, out_hbm, idx_v, rows_v, sem)


def _dense_body(e_ref, we_ref, w1a_ref, w1g_ref, w2_ref, w3_ref, w4_ref,
                lrw_ref, b1_ref, b2_ref, b3_ref, bo_ref, out_ref):
    x = e_ref[...]
    bb = x.shape[0]
    e3 = x.reshape(bb, _F, _D)
    gram = lax.dot_general(
        e3, e3, (((2,), (2,)), ((0,), (0,))),
        preferred_element_type=jnp.float32)
    gflat = gram.reshape(bb, _F * _F)
    h = x @ w1a_ref[...] + gflat @ w1g_ref[...] + b1_ref[...]
    h = jnp.maximum(h, 0.0)
    h = jnp.maximum(h @ w2_ref[...] + b2_ref[...], 0.0)
    h = jnp.maximum(h @ w3_ref[...] + b3_ref[...], 0.0)
    logit = h @ w4_ref[...] + we_ref[...] @ lrw_ref[...] + bo_ref[...]
    out_ref[...] = jax.nn.sigmoid(logit)


def _dense_call(e, we, w1a, w1g, w2, w3, w4, lrw, b1, b2, b3, bo, bB=512):
    grid = _B // bB
    fd = _F * _D
    return pl.pallas_call(
        _dense_body,
        grid=(grid,),
        in_specs=[
            pl.BlockSpec((bB, fd), lambda i: (i, 0)),
            pl.BlockSpec((bB, fd), lambda i: (i, 0)),
            pl.BlockSpec((fd, 512), lambda i: (0, 0)),
            pl.BlockSpec((_F * _F, 512), lambda i: (0, 0)),
            pl.BlockSpec((512, 512), lambda i: (0, 0)),
            pl.BlockSpec((512, 512), lambda i: (0, 0)),
            pl.BlockSpec((512, 1), lambda i: (0, 0)),
            pl.BlockSpec((fd, 1), lambda i: (0, 0)),
            pl.BlockSpec((1, 512), lambda i: (0, 0)),
            pl.BlockSpec((1, 512), lambda i: (0, 0)),
            pl.BlockSpec((1, 512), lambda i: (0, 0)),
            pl.BlockSpec((1, 1), lambda i: (0, 0)),
        ],
        out_specs=pl.BlockSpec((bB, 1), lambda i: (i, 0)),
        out_shape=jax.ShapeDtypeStruct((_B, 1), jnp.float32),
    )(e, we, w1a, w1g, w2, w3, w4, lrw, b1, b2, b3, bo)


def kernel(inputs, deep_table, wide_table, W1, b1, W2, b2, W3, b3, W4, b4, lr_W, lr_b):
    idx = inputs.reshape(_TOT).astype(jnp.int32)
    e = _sc_gather(deep_table, idx).reshape(_B, _F * _D)
    we = _sc_gather(wide_table, idx).reshape(_B, _F * _D)

    iu0, iu1 = np.triu_indices(_F, k=1)
    w1a = W1[: _F * _D]
    w1b = W1[_F * _D :]
    w1g = jnp.zeros((_F * _F, 512), jnp.float32).at[iu0 * _F + iu1].set(w1b)

    bo = (b4 + lr_b).reshape(1, 1)
    out = _dense_call(
        e, we, w1a, w1g, W2, W3, W4, lr_W,
        b1.reshape(1, 512), b2.reshape(1, 512), b3.reshape(1, 512), bo)
    return out
